# baseline (speedup 1.0000x reference)
import functools

import jax
import jax.numpy as jnp
from jax import lax
from jax.experimental import pallas as pl
from jax.experimental.pallas import tpu as pltpu
from jax.experimental.pallas import tpu_sc as plsc

NUM_CORES = 2
NUM_SUBCORES = 16
LANES = 16
NUM_WORKERS = NUM_CORES * NUM_SUBCORES


def _make_kernel(batch, dim):
    assert batch % (8 * NUM_WORKERS) == 0
    bpw = batch // NUM_WORKERS
    groups = bpw // LANES
    mesh = plsc.VectorSubcoreMesh(
        core_axis_name="c", subcore_axis_name="s"
    )

    @functools.partial(
        pl.kernel,
        out_type=jax.ShapeDtypeStruct((batch,), jnp.float32),
        mesh=mesh,
        compiler_params=pltpu.CompilerParams(
            needs_layout_passes=False, use_tc_tiling_on_sc=False
        ),
        scratch_types=[
            pltpu.VMEM((bpw,), jnp.int32),
            pltpu.VMEM((bpw,), jnp.int32),
            pltpu.VMEM((bpw, dim), jnp.float32),
            pltpu.VMEM((bpw, dim), jnp.float32),
            pltpu.VMEM((bpw,), jnp.float32),
            pltpu.SemaphoreType.DMA,
            pltpu.SemaphoreType.DMA,
        ],
    )
    def sc_kernel(movies_hbm, users_hbm, mtab_hbm, utab_hbm, out_hbm,
                  midx_v, uidx_v, em_v, eu_v, outv, sem_m, sem_u):
        wid = lax.axis_index("s") * NUM_CORES + lax.axis_index("c")
        base = wid * bpw

        pltpu.sync_copy(movies_hbm.at[pl.ds(base, bpw)], midx_v)
        pltpu.sync_copy(users_hbm.at[pl.ds(base, bpw)], uidx_v)

        cm = pltpu.async_copy(mtab_hbm.at[midx_v], em_v, sem_m)
        cu = pltpu.async_copy(utab_hbm.at[uidx_v], eu_v, sem_u)
        cm.wait()
        cu.wait()

        zeros = jnp.zeros((LANES,), jnp.float32)

        def zero_body(g, _):
            outv[pl.ds(g * LANES, LANES)] = zeros
            return 0

        lax.fori_loop(0, groups, zero_body, 0)

        half = dim // 2

        def group_body(g, _):
            b0 = g * LANES
            for j in range(LANES):
                b = b0 + j
                em_lo = em_v[b, pl.ds(0, half)]
                em_hi = em_v[b, pl.ds(half, half)]
                eu_lo = eu_v[b, pl.ds(0, half)]
                eu_hi = eu_v[b, pl.ds(half, half)]
                part = em_lo * eu_lo + em_hi * eu_hi
                plsc.addupdate_scatter(
                    outv, [jnp.zeros((LANES,), jnp.int32) + b], part
                )
            return 0

        lax.fori_loop(0, groups, group_body, 0)

        pltpu.sync_copy(outv, out_hbm.at[pl.ds(base, bpw)])

    return sc_kernel


@jax.jit
def kernel(movies, users, movie_table, user_table):
    batch = movies.shape[0]
    dim = movie_table.shape[1]
    out = _make_kernel(batch, dim)(
        movies.astype(jnp.int32), users.astype(jnp.int32),
        movie_table, user_table
    )
    return out.reshape(batch, 1)
